# baseline (speedup 1.0000x reference)
import functools

import jax
import jax.numpy as jnp
from jax import lax
from jax.experimental import pallas as pl
from jax.experimental.pallas import tpu as pltpu
from jax.experimental.pallas import tpu_sc as plsc

_N = 10000
_E = 320000
_D = 128

_K = 80
_NCHUNKS = _E // _K
_NC = 2
_NS = 16
_NT = _NC * _NS
_CPT = _NCHUNKS // _NT
_RBUF = 4
_IBUF = 8
_NP = 10112
_RPT = _NP // _NS

_mesh = plsc.VectorSubcoreMesh(core_axis_name="c", subcore_axis_name="s")


@functools.partial(
    pl.kernel,
    out_type=jax.ShapeDtypeStruct((_NC, _NP), jnp.float32),
    mesh=_mesh,
    scratch_types=[
        pltpu.VMEM((8, _K), jnp.int32),
        pltpu.VMEM((_K,), jnp.float32),
        pltpu.VMEM((640,), jnp.float32),
        pltpu.VMEM_SHARED((_NP,), jnp.float32),
    ]
    + [pltpu.SemaphoreType.DMA] * 12,
)
def _sc_count(dst4_hbm, out_hbm, idx_v, ones_v, zbuf, acc, *sems):
    isem = sems[:8]
    csem = sems[8:]
    c = lax.axis_index("c")
    s = lax.axis_index("s")
    t = c * _NS + s
    c0 = t * _CPT

    def setv(i, _):
        ones_v[pl.ds(i * 16, 16)] = jnp.ones((16,), jnp.float32)
        return 0

    lax.fori_loop(0, _K // 16, setv, 0)

    def zr(i, _):
        zbuf[pl.ds(i * 16, 16)] = jnp.zeros((16,), jnp.float32)
        return 0

    lax.fori_loop(0, 640 // 16, zr, 0)

    @pl.when(s < _NS - 1)
    def _zmain():
        pltpu.sync_copy(zbuf, acc.at[pl.ds(s * 640, 640)])

    @pl.when(s == _NS - 1)
    def _ztail():
        pltpu.sync_copy(zbuf.at[pl.ds(0, 512)], acc.at[pl.ds(9600, 512)])

    def ifetch(cl, q):
        pltpu.async_copy(dst4_hbm.at[c0 + cl], idx_v.at[pl.ds(q, 1)], isem[q])

    def iwait(cl, q):
        pltpu.make_async_copy(
            dst4_hbm.at[c0 + cl], idx_v.at[pl.ds(q, 1)], isem[q]).wait()

    def cstart(q8, j4):
        pltpu.async_copy(ones_v, acc.at[idx_v.at[q8]], csem[j4], add=True)

    def cwait(j4):
        pltpu.make_async_copy(ones_v, acc.at[idx_v.at[0]], csem[j4]).wait()

    for u in range(4):
        ifetch(u, u)
    plsc.subcore_barrier()

    def chunk(cl, u, first=False):
        q8 = u % 8
        j4 = u % 4
        q4n = (u + 4) % 8
        iwait(cl, q8)
        if not first:
            cwait(j4)
        cstart(q8, j4)
        if isinstance(cl, int):
            if cl + 4 < _CPT:
                ifetch(cl + 4, q4n)
        else:
            @pl.when(cl + 4 < _CPT)
            def _pf():
                ifetch(cl + 4, q4n)

    for cl in range(4):
        chunk(cl, cl, first=True)

    def body(j, _):
        for k in range(8):
            u = (4 + k) % 8
            chunk(4 + j * 8 + k, u)
        return 0

    lax.fori_loop(0, 15, body, 0)
    chunk(124, 124 % 8)

    for u in (1, 2, 3, 0):
        cwait(u)

    plsc.subcore_barrier()

    @pl.when(s == 0)
    def _out():
        pltpu.sync_copy(acc, out_hbm.at[c])


@functools.partial(
    pl.kernel,
    out_type=jax.ShapeDtypeStruct((_NC, _NP, _D), jnp.float32),
    mesh=_mesh,
    scratch_types=[
        pltpu.VMEM((_IBUF, _K), jnp.int32),
        pltpu.VMEM((_IBUF, _K), jnp.int32),
        pltpu.VMEM((_RBUF, _K, _D), jnp.float32),
        pltpu.VMEM_SHARED((_NP, _D), jnp.float32),
    ]
    + [pltpu.SemaphoreType.DMA] * (2 * _IBUF + 2 * _RBUF),
)
def _sc_agg(y_hbm, src4_hbm, dst4_hbm, out_hbm, src_v, dst_v, rows_v, acc,
            *sems):
    ssem = sems[:_IBUF]
    dsem = sems[_IBUF:2 * _IBUF]
    gsem = sems[2 * _IBUF:2 * _IBUF + _RBUF]
    zsem = sems[2 * _IBUF + _RBUF:]
    c = lax.axis_index("c")
    s = lax.axis_index("s")
    t = c * _NS + s
    c0 = t * _CPT

    def zrow(i, _):
        for j in range(_D // 16):
            rows_v[0, i, pl.ds(j * 16, 16)] = jnp.zeros((16,), jnp.float32)
        return 0

    lax.fori_loop(0, _K, zrow, 0)
    base = s * _RPT
    for i in range(7):
        pltpu.sync_copy(rows_v.at[0], acc.at[pl.ds(base + i * _K, _K)])
    pltpu.sync_copy(rows_v.at[0, pl.ds(0, 72)], acc.at[pl.ds(base + 560, 72)])

    def ifetch(cl, q):
        pltpu.async_copy(src4_hbm.at[c0 + cl], src_v.at[pl.ds(q, 1)], ssem[q])
        pltpu.async_copy(dst4_hbm.at[c0 + cl], dst_v.at[pl.ds(q, 1)], dsem[q])

    def iwait(cl, q):
        pltpu.make_async_copy(
            src4_hbm.at[c0 + cl], src_v.at[pl.ds(q, 1)], ssem[q]).wait()
        pltpu.make_async_copy(
            dst4_hbm.at[c0 + cl], dst_v.at[pl.ds(q, 1)], dsem[q]).wait()

    def gfetch(q, r):
        pltpu.async_copy(y_hbm.at[src_v.at[q]], rows_v.at[r], gsem[r])

    def gwait(q, r):
        pltpu.make_async_copy(
            y_hbm.at[src_v.at[q]], rows_v.at[r], gsem[r]).wait()

    def zstart(q, r):
        pltpu.async_copy(rows_v.at[r], acc.at[dst_v.at[q]], zsem[r], add=True)

    def zwait(q, r):
        pltpu.make_async_copy(
            rows_v.at[r], acc.at[dst_v.at[q]], zsem[r]).wait()

    for u in range(6):
        ifetch(u, u)
    for u in range(3):
        iwait(u, u)
        gfetch(u, u)
    plsc.subcore_barrier()

    def chunk(cl, u, first=False):
        r = u % _RBUF
        q3 = (u + 3) % _IBUF
        r3 = (u + 3) % _RBUF
        q6 = (u + 6) % _IBUF
        gwait(u, r)
        if not first:
            zwait(q3, r3)
        zstart(u, r)
        if isinstance(cl, int):
            if cl + 6 < _CPT:
                ifetch(cl + 6, q6)
            if cl + 3 < _CPT:
                iwait(cl + 3, q3)
                gfetch(q3, r3)
        else:
            @pl.when(cl + 6 < _CPT)
            def _pf():
                ifetch(cl + 6, q6)

            iwait(cl + 3, q3)
            gfetch(q3, r3)

    chunk(0, 0, first=True)

    def body(j, _):
        for k in range(_IBUF):
            u = (1 + k) % _IBUF
            chunk(1 + j * _IBUF + k, u)
        return 0

    lax.fori_loop(0, 15, body, 0)
    for cl in range(121, _CPT):
        chunk(cl, cl % _IBUF)

    zwait(124 % _IBUF, 124 % _RBUF)

    plsc.subcore_barrier()
    pltpu.sync_copy(acc.at[pl.ds(base, _RPT)], out_hbm.at[c, pl.ds(base, _RPT)])


_GRID = 10
_BR = _N // _GRID


def _tc_prep_body(x_ref, w_ref, cnt_ref, y_ref, dinv_ref):
    deg = cnt_ref[0] + cnt_ref[1] + 1.0
    dinv = lax.rsqrt(deg)
    h = jnp.dot(x_ref[...], w_ref[...], preferred_element_type=jnp.float32)
    y_ref[...] = h * dinv
    dinv_ref[...] = dinv


_tc_prep = pl.pallas_call(
    _tc_prep_body,
    grid=(_GRID,),
    in_specs=[
        pl.BlockSpec((_BR, _D), lambda i: (i, 0)),
        pl.BlockSpec((_D, _D), lambda i: (0, 0)),
        pl.BlockSpec((_NC, _BR, 1), lambda i: (0, i, 0)),
    ],
    out_specs=(
        pl.BlockSpec((_BR, _D), lambda i: (i, 0)),
        pl.BlockSpec((_BR, 1), lambda i: (i, 0)),
    ),
    out_shape=(
        jax.ShapeDtypeStruct((_N, _D), jnp.float32),
        jax.ShapeDtypeStruct((_N, 1), jnp.float32),
    ),
)


def _tc_mid_body(p_ref, y1_ref, dinv_ref, b_ref, w_ref, y2_ref):
    agg = p_ref[0] + p_ref[1] + y1_ref[...]
    out1 = dinv_ref[...] * agg + b_ref[...]
    h2 = jnp.dot(out1, w_ref[...], preferred_element_type=jnp.float32)
    y2_ref[...] = h2 * dinv_ref[...]


_tc_mid = pl.pallas_call(
    _tc_mid_body,
    grid=(_GRID,),
    in_specs=[
        pl.BlockSpec((_NC, _BR, _D), lambda i: (0, i, 0)),
        pl.BlockSpec((_BR, _D), lambda i: (i, 0)),
        pl.BlockSpec((_BR, 1), lambda i: (i, 0)),
        pl.BlockSpec((1, _D), lambda i: (0, 0)),
        pl.BlockSpec((_D, _D), lambda i: (0, 0)),
    ],
    out_specs=pl.BlockSpec((_BR, _D), lambda i: (i, 0)),
    out_shape=jax.ShapeDtypeStruct((_N, _D), jnp.float32),
)


def _tc_fin_body(q_ref, y2_ref, dinv_ref, b_ref, o_ref):
    z = dinv_ref[...] * (q_ref[0] + q_ref[1] + y2_ref[...]) + b_ref[...]
    o_ref[...] = jnp.where(z >= 0, z, 0.1 * z)


_tc_fin = pl.pallas_call(
    _tc_fin_body,
    grid=(_GRID,),
    in_specs=[
        pl.BlockSpec((_NC, _BR, _D), lambda i: (0, i, 0)),
        pl.BlockSpec((_BR, _D), lambda i: (i, 0)),
        pl.BlockSpec((_BR, 1), lambda i: (i, 0)),
        pl.BlockSpec((1, _D), lambda i: (0, 0)),
    ],
    out_specs=pl.BlockSpec((_BR, _D), lambda i: (i, 0)),
    out_shape=jax.ShapeDtypeStruct((_N, _D), jnp.float32),
)


@jax.jit
def _run(x, edge_index, W1, b1, W2, b2):
    src4 = edge_index[0].reshape(_NCHUNKS, 1, _K)
    dst4 = edge_index[1].reshape(_NCHUNKS, 1, _K)
    cnt = _sc_count(dst4).reshape(_NC, _NP, 1)
    y1, dinv = _tc_prep(x, W1, cnt)
    p = _sc_agg(y1, src4, dst4)
    y2 = _tc_mid(p, y1, dinv, b1.reshape(1, _D), W2)
    q = _sc_agg(y2, src4, dst4)
    return _tc_fin(q, y2, dinv, b2.reshape(1, _D))


def kernel(x, edge_index, W1, b1, W2, b2):
    return _run(x, edge_index, W1, b1, W2, b2)


# speedup vs baseline: 1.0217x; 1.0217x over previous
import functools

import jax
import jax.numpy as jnp
from jax import lax
from jax.experimental import pallas as pl
from jax.experimental.pallas import tpu as pltpu
from jax.experimental.pallas import tpu_sc as plsc

_N = 10000
_E = 320000
_D = 128

_K = 80
_NCHUNKS = _E // _K
_NC = 2
_NS = 16
_NT = _NC * _NS
_CPT = _NCHUNKS // _NT
_RBUF = 4
_IBUF = 8
_NP = 10112
_RPT = _NP // _NS

_mesh = plsc.VectorSubcoreMesh(core_axis_name="c", subcore_axis_name="s")


@functools.partial(
    pl.kernel,
    out_type=jax.ShapeDtypeStruct((_NC, _NP), jnp.float32),
    mesh=_mesh,
    scratch_types=[
        pltpu.VMEM((8, _K), jnp.int32),
        pltpu.VMEM((_K,), jnp.float32),
        pltpu.VMEM((640,), jnp.float32),
        pltpu.VMEM_SHARED((_NP,), jnp.float32),
    ]
    + [pltpu.SemaphoreType.DMA] * 12,
)
def _sc_count(dst4_hbm, out_hbm, idx_v, ones_v, zbuf, acc, *sems):
    isem = sems[:8]
    csem = sems[8:]
    c = lax.axis_index("c")
    s = lax.axis_index("s")
    t = c * _NS + s
    c0 = t * _CPT

    def setv(i, _):
        ones_v[pl.ds(i * 16, 16)] = jnp.ones((16,), jnp.float32)
        return 0

    lax.fori_loop(0, _K // 16, setv, 0)

    def zr(i, _):
        zbuf[pl.ds(i * 16, 16)] = jnp.zeros((16,), jnp.float32)
        return 0

    lax.fori_loop(0, 640 // 16, zr, 0)

    @pl.when(s < _NS - 1)
    def _zmain():
        pltpu.sync_copy(zbuf, acc.at[pl.ds(s * 640, 640)])

    @pl.when(s == _NS - 1)
    def _ztail():
        pltpu.sync_copy(zbuf.at[pl.ds(0, 512)], acc.at[pl.ds(9600, 512)])

    def ifetch(cl, q):
        pltpu.async_copy(dst4_hbm.at[c0 + cl], idx_v.at[pl.ds(q, 1)], isem[q])

    def iwait(cl, q):
        pltpu.make_async_copy(
            dst4_hbm.at[c0 + cl], idx_v.at[pl.ds(q, 1)], isem[q]).wait()

    def cstart(q8, j4):
        pltpu.async_copy(ones_v, acc.at[idx_v.at[q8]], csem[j4], add=True)

    def cwait(j4):
        pltpu.make_async_copy(ones_v, acc.at[idx_v.at[0]], csem[j4]).wait()

    for u in range(4):
        ifetch(u, u)
    plsc.subcore_barrier()

    def chunk(cl, u, first=False):
        q8 = u % 8
        j4 = u % 4
        q4n = (u + 4) % 8
        iwait(cl, q8)
        if not first:
            cwait(j4)
        cstart(q8, j4)
        if isinstance(cl, int):
            if cl + 4 < _CPT:
                ifetch(cl + 4, q4n)
        else:
            @pl.when(cl + 4 < _CPT)
            def _pf():
                ifetch(cl + 4, q4n)

    for cl in range(4):
        chunk(cl, cl, first=True)

    def body(j, _):
        for k in range(8):
            u = (4 + k) % 8
            chunk(4 + j * 8 + k, u)
        return 0

    lax.fori_loop(0, 15, body, 0)
    chunk(124, 124 % 8)

    for u in (1, 2, 3, 0):
        cwait(u)

    plsc.subcore_barrier()

    @pl.when(s == 0)
    def _out():
        pltpu.sync_copy(acc, out_hbm.at[c])


@functools.partial(
    pl.kernel,
    out_type=jax.ShapeDtypeStruct((_NC, _NP, _D), jnp.float32),
    mesh=_mesh,
    scratch_types=[
        pltpu.VMEM((_IBUF, _K), jnp.int32),
        pltpu.VMEM((_IBUF, _K), jnp.int32),
        pltpu.VMEM((_RBUF, _K, _D), jnp.float32),
        pltpu.VMEM_SHARED((_NP, _D), jnp.float32),
    ]
    + [pltpu.SemaphoreType.DMA] * (2 * _IBUF + 2 * _RBUF),
)
def _sc_agg(y_hbm, src4_hbm, dst4_hbm, out_hbm, src_v, dst_v, rows_v, acc,
            *sems):
    ssem = sems[:_IBUF]
    dsem = sems[_IBUF:2 * _IBUF]
    gsem = sems[2 * _IBUF:2 * _IBUF + _RBUF]
    zsem = sems[2 * _IBUF + _RBUF:]
    c = lax.axis_index("c")
    s = lax.axis_index("s")
    t = c * _NS + s
    c0 = t * _CPT

    def zrow(i, _):
        for j in range(_D // 16):
            rows_v[0, i, pl.ds(j * 16, 16)] = jnp.zeros((16,), jnp.float32)
        return 0

    lax.fori_loop(0, _K, zrow, 0)
    base = s * _RPT
    for i in range(7):
        pltpu.sync_copy(rows_v.at[0], acc.at[pl.ds(base + i * _K, _K)])
    pltpu.sync_copy(rows_v.at[0, pl.ds(0, 72)], acc.at[pl.ds(base + 560, 72)])

    def ifetch(cl, q):
        pltpu.async_copy(src4_hbm.at[c0 + cl], src_v.at[pl.ds(q, 1)], ssem[q])
        pltpu.async_copy(dst4_hbm.at[c0 + cl], dst_v.at[pl.ds(q, 1)], dsem[q])

    def iwait(cl, q):
        pltpu.make_async_copy(
            src4_hbm.at[c0 + cl], src_v.at[pl.ds(q, 1)], ssem[q]).wait()
        pltpu.make_async_copy(
            dst4_hbm.at[c0 + cl], dst_v.at[pl.ds(q, 1)], dsem[q]).wait()

    def gfetch(q, r):
        pltpu.async_copy(y_hbm.at[src_v.at[q]], rows_v.at[r], gsem[r])

    def gwait(q, r):
        pltpu.make_async_copy(
            y_hbm.at[src_v.at[q]], rows_v.at[r], gsem[r]).wait()

    def zstart(q, r):
        pltpu.async_copy(rows_v.at[r], acc.at[dst_v.at[q]], zsem[r], add=True)

    def zwait(q, r):
        pltpu.make_async_copy(
            rows_v.at[r], acc.at[dst_v.at[q]], zsem[r]).wait()

    for u in range(6):
        ifetch(u, u)
    for u in range(3):
        iwait(u, u)
        gfetch(u, u)
    plsc.subcore_barrier()

    def chunk(cl, u, first=False):
        r = u % _RBUF
        q3 = (u + 3) % _IBUF
        r3 = (u + 3) % _RBUF
        q6 = (u + 6) % _IBUF
        gwait(u, r)
        if not first:
            zwait(q3, r3)
        zstart(u, r)
        if isinstance(cl, int):
            if cl + 6 < _CPT:
                ifetch(cl + 6, q6)
            if cl + 3 < _CPT:
                iwait(cl + 3, q3)
                gfetch(q3, r3)
        else:
            @pl.when(cl + 6 < _CPT)
            def _pf():
                ifetch(cl + 6, q6)

            iwait(cl + 3, q3)
            gfetch(q3, r3)

    chunk(0, 0, first=True)

    def body(j, _):
        for k in range(_IBUF):
            u = (1 + k) % _IBUF
            chunk(1 + j * _IBUF + k, u)
        return 0

    lax.fori_loop(0, 15, body, 0)
    for cl in range(121, _CPT):
        chunk(cl, cl % _IBUF)

    zwait(124 % _IBUF, 124 % _RBUF)

    plsc.subcore_barrier()
    pltpu.sync_copy(acc.at[pl.ds(base, _RPT)], out_hbm.at[c, pl.ds(base, _RPT)])


_GRID = 5
_BR = _N // _GRID


def _tc_prep_body(x_ref, w_ref, cnt_ref, y_ref, dinv_ref):
    deg = cnt_ref[0] + cnt_ref[1] + 1.0
    dinv = lax.rsqrt(deg)
    h = jnp.dot(x_ref[...], w_ref[...], preferred_element_type=jnp.float32)
    y_ref[...] = h * dinv
    dinv_ref[...] = dinv


_tc_prep = pl.pallas_call(
    _tc_prep_body,
    grid=(_GRID,),
    in_specs=[
        pl.BlockSpec((_BR, _D), lambda i: (i, 0)),
        pl.BlockSpec((_D, _D), lambda i: (0, 0)),
        pl.BlockSpec((_NC, _BR, 1), lambda i: (0, i, 0)),
    ],
    out_specs=(
        pl.BlockSpec((_BR, _D), lambda i: (i, 0)),
        pl.BlockSpec((_BR, 1), lambda i: (i, 0)),
    ),
    out_shape=(
        jax.ShapeDtypeStruct((_N, _D), jnp.float32),
        jax.ShapeDtypeStruct((_N, 1), jnp.float32),
    ),
)


def _tc_mid_body(p_ref, y1_ref, dinv_ref, b_ref, w_ref, y2_ref):
    agg = p_ref[0] + p_ref[1] + y1_ref[...]
    out1 = dinv_ref[...] * agg + b_ref[...]
    h2 = jnp.dot(out1, w_ref[...], preferred_element_type=jnp.float32)
    y2_ref[...] = h2 * dinv_ref[...]


_tc_mid = pl.pallas_call(
    _tc_mid_body,
    grid=(_GRID,),
    in_specs=[
        pl.BlockSpec((_NC, _BR, _D), lambda i: (0, i, 0)),
        pl.BlockSpec((_BR, _D), lambda i: (i, 0)),
        pl.BlockSpec((_BR, 1), lambda i: (i, 0)),
        pl.BlockSpec((1, _D), lambda i: (0, 0)),
        pl.BlockSpec((_D, _D), lambda i: (0, 0)),
    ],
    out_specs=pl.BlockSpec((_BR, _D), lambda i: (i, 0)),
    out_shape=jax.ShapeDtypeStruct((_N, _D), jnp.float32),
)


def _tc_fin_body(q_ref, y2_ref, dinv_ref, b_ref, o_ref):
    z = dinv_ref[...] * (q_ref[0] + q_ref[1] + y2_ref[...]) + b_ref[...]
    o_ref[...] = jnp.where(z >= 0, z, 0.1 * z)


_tc_fin = pl.pallas_call(
    _tc_fin_body,
    grid=(_GRID,),
    in_specs=[
        pl.BlockSpec((_NC, _BR, _D), lambda i: (0, i, 0)),
        pl.BlockSpec((_BR, _D), lambda i: (i, 0)),
        pl.BlockSpec((_BR, 1), lambda i: (i, 0)),
        pl.BlockSpec((1, _D), lambda i: (0, 0)),
    ],
    out_specs=pl.BlockSpec((_BR, _D), lambda i: (i, 0)),
    out_shape=jax.ShapeDtypeStruct((_N, _D), jnp.float32),
)


@jax.jit
def _run(x, edge_index, W1, b1, W2, b2):
    src4 = edge_index[0].reshape(_NCHUNKS, 1, _K)
    dst4 = edge_index[1].reshape(_NCHUNKS, 1, _K)
    cnt = _sc_count(dst4).reshape(_NC, _NP, 1)
    y1, dinv = _tc_prep(x, W1, cnt)
    p = _sc_agg(y1, src4, dst4)
    y2 = _tc_mid(p, y1, dinv, b1.reshape(1, _D), W2)
    q = _sc_agg(y2, src4, dst4)
    return _tc_fin(q, y2, dinv, b2.reshape(1, _D))


def kernel(x, edge_index, W1, b1, W2, b2):
    return _run(x, edge_index, W1, b1, W2, b2)


# speedup vs baseline: 1.0292x; 1.0073x over previous
import functools

import jax
import jax.numpy as jnp
from jax import lax
from jax.experimental import pallas as pl
from jax.experimental.pallas import tpu as pltpu
from jax.experimental.pallas import tpu_sc as plsc

_N = 10000
_E = 320000
_D = 128

_K = 80
_NCHUNKS = _E // _K
_NC = 2
_NS = 16
_NT = _NC * _NS
_CPT = _NCHUNKS // _NT
_RBUF = 4
_IBUF = 8
_NP = 10112
_RPT = _NP // _NS

_mesh = plsc.VectorSubcoreMesh(core_axis_name="c", subcore_axis_name="s")


@functools.partial(
    pl.kernel,
    out_type=jax.ShapeDtypeStruct((_NC, _NP), jnp.float32),
    mesh=_mesh,
    scratch_types=[
        pltpu.VMEM((8, _K), jnp.int32),
        pltpu.VMEM((_K,), jnp.float32),
        pltpu.VMEM((640,), jnp.float32),
        pltpu.VMEM_SHARED((_NP,), jnp.float32),
    ]
    + [pltpu.SemaphoreType.DMA] * 12,
)
def _sc_count(dst4_hbm, out_hbm, idx_v, ones_v, zbuf, acc, *sems):
    isem = sems[:8]
    csem = sems[8:]
    c = lax.axis_index("c")
    s = lax.axis_index("s")
    t = c * _NS + s
    c0 = t * _CPT

    def setv(i, _):
        ones_v[pl.ds(i * 16, 16)] = jnp.ones((16,), jnp.float32)
        return 0

    lax.fori_loop(0, _K // 16, setv, 0)

    def zr(i, _):
        zbuf[pl.ds(i * 16, 16)] = jnp.zeros((16,), jnp.float32)
        return 0

    lax.fori_loop(0, 640 // 16, zr, 0)

    @pl.when(s < _NS - 1)
    def _zmain():
        pltpu.sync_copy(zbuf, acc.at[pl.ds(s * 640, 640)])

    @pl.when(s == _NS - 1)
    def _ztail():
        pltpu.sync_copy(zbuf.at[pl.ds(0, 512)], acc.at[pl.ds(9600, 512)])

    def ifetch(cl, q):
        pltpu.async_copy(dst4_hbm.at[c0 + cl], idx_v.at[pl.ds(q, 1)], isem[q])

    def iwait(cl, q):
        pltpu.make_async_copy(
            dst4_hbm.at[c0 + cl], idx_v.at[pl.ds(q, 1)], isem[q]).wait()

    def cstart(q8, j4):
        pltpu.async_copy(ones_v, acc.at[idx_v.at[q8]], csem[j4], add=True)

    def cwait(j4):
        pltpu.make_async_copy(ones_v, acc.at[idx_v.at[0]], csem[j4]).wait()

    for u in range(4):
        ifetch(u, u)
    plsc.subcore_barrier()

    def chunk(cl, u, first=False):
        q8 = u % 8
        j4 = u % 4
        q4n = (u + 4) % 8
        iwait(cl, q8)
        if not first:
            cwait(j4)
        cstart(q8, j4)
        if isinstance(cl, int):
            if cl + 4 < _CPT:
                ifetch(cl + 4, q4n)
        else:
            @pl.when(cl + 4 < _CPT)
            def _pf():
                ifetch(cl + 4, q4n)

    for cl in range(4):
        chunk(cl, cl, first=True)

    def body(j, _):
        for k in range(8):
            u = (4 + k) % 8
            chunk(4 + j * 8 + k, u)
        return 0

    lax.fori_loop(0, 15, body, 0)
    chunk(124, 124 % 8)

    for u in (1, 2, 3, 0):
        cwait(u)

    plsc.subcore_barrier()

    @pl.when(s == 0)
    def _out():
        pltpu.sync_copy(acc, out_hbm.at[c])


@functools.partial(
    pl.kernel,
    out_type=jax.ShapeDtypeStruct((_NC, _NP, _D), jnp.float32),
    mesh=_mesh,
    scratch_types=[
        pltpu.VMEM((_IBUF, _K), jnp.int32),
        pltpu.VMEM((_IBUF, _K), jnp.int32),
        pltpu.VMEM((_RBUF, _K, _D), jnp.float32),
        pltpu.VMEM_SHARED((_NP, _D), jnp.float32),
    ]
    + [pltpu.SemaphoreType.DMA] * (2 * _IBUF + 2 * _RBUF),
)
def _sc_agg(y_hbm, src4_hbm, dst4_hbm, out_hbm, src_v, dst_v, rows_v, acc,
            *sems):
    ssem = sems[:_IBUF]
    dsem = sems[_IBUF:2 * _IBUF]
    gsem = sems[2 * _IBUF:2 * _IBUF + _RBUF]
    zsem = sems[2 * _IBUF + _RBUF:]
    c = lax.axis_index("c")
    s = lax.axis_index("s")
    t = c * _NS + s
    c0 = t * _CPT

    def zrow(i, _):
        for j in range(_D // 16):
            rows_v[0, i, pl.ds(j * 16, 16)] = jnp.zeros((16,), jnp.float32)
        return 0

    lax.fori_loop(0, _K, zrow, 0)
    base = s * _RPT
    for i in range(7):
        pltpu.sync_copy(rows_v.at[0], acc.at[pl.ds(base + i * _K, _K)])
    pltpu.sync_copy(rows_v.at[0, pl.ds(0, 72)], acc.at[pl.ds(base + 560, 72)])

    def ifetch(cl, q):
        pltpu.async_copy(src4_hbm.at[c0 + cl], src_v.at[pl.ds(q, 1)], ssem[q])
        pltpu.async_copy(dst4_hbm.at[c0 + cl], dst_v.at[pl.ds(q, 1)], dsem[q])

    def iwait(cl, q):
        pltpu.make_async_copy(
            src4_hbm.at[c0 + cl], src_v.at[pl.ds(q, 1)], ssem[q]).wait()
        pltpu.make_async_copy(
            dst4_hbm.at[c0 + cl], dst_v.at[pl.ds(q, 1)], dsem[q]).wait()

    def gfetch(q, r):
        pltpu.async_copy(y_hbm.at[src_v.at[q]], rows_v.at[r], gsem[r])

    def gwait(q, r):
        pltpu.make_async_copy(
            y_hbm.at[src_v.at[q]], rows_v.at[r], gsem[r]).wait()

    def zstart(q, r):
        pltpu.async_copy(rows_v.at[r], acc.at[dst_v.at[q]], zsem[r], add=True)

    def zwait(q, r):
        pltpu.make_async_copy(
            rows_v.at[r], acc.at[dst_v.at[q]], zsem[r]).wait()

    for u in range(6):
        ifetch(u, u)
    for u in range(3):
        iwait(u, u)
        gfetch(u, u)
    plsc.subcore_barrier()

    def chunk(cl, u, first=False):
        r = u % _RBUF
        q3 = (u + 3) % _IBUF
        r3 = (u + 3) % _RBUF
        q6 = (u + 6) % _IBUF
        gwait(u, r)
        if not first:
            zwait(q3, r3)
        zstart(u, r)
        if isinstance(cl, int):
            if cl + 6 < _CPT:
                ifetch(cl + 6, q6)
            if cl + 3 < _CPT:
                iwait(cl + 3, q3)
                gfetch(q3, r3)
        else:
            @pl.when(cl + 6 < _CPT)
            def _pf():
                ifetch(cl + 6, q6)

            iwait(cl + 3, q3)
            gfetch(q3, r3)

    chunk(0, 0, first=True)

    def body(j, _):
        for k in range(_IBUF):
            u = (1 + k) % _IBUF
            chunk(1 + j * _IBUF + k, u)
        return 0

    lax.fori_loop(0, 15, body, 0)
    for cl in range(121, _CPT):
        chunk(cl, cl % _IBUF)

    zwait(124 % _IBUF, 124 % _RBUF)

    plsc.subcore_barrier()
    pltpu.sync_copy(acc.at[pl.ds(base, _RPT)], out_hbm.at[c, pl.ds(base, _RPT)])


_GRID = 2
_BR = _N // _GRID


def _tc_prep_body(x_ref, w_ref, cnt_ref, y_ref, dinv_ref):
    deg = cnt_ref[0] + cnt_ref[1] + 1.0
    dinv = lax.rsqrt(deg)
    h = jnp.dot(x_ref[...], w_ref[...], preferred_element_type=jnp.float32)
    y_ref[...] = h * dinv
    dinv_ref[...] = dinv


_tc_prep = pl.pallas_call(
    _tc_prep_body,
    grid=(_GRID,),
    in_specs=[
        pl.BlockSpec((_BR, _D), lambda i: (i, 0)),
        pl.BlockSpec((_D, _D), lambda i: (0, 0)),
        pl.BlockSpec((_NC, _BR, 1), lambda i: (0, i, 0)),
    ],
    out_specs=(
        pl.BlockSpec((_BR, _D), lambda i: (i, 0)),
        pl.BlockSpec((_BR, 1), lambda i: (i, 0)),
    ),
    out_shape=(
        jax.ShapeDtypeStruct((_N, _D), jnp.float32),
        jax.ShapeDtypeStruct((_N, 1), jnp.float32),
    ),
)


def _tc_mid_body(p_ref, y1_ref, dinv_ref, b_ref, w_ref, y2_ref):
    agg = p_ref[0] + p_ref[1] + y1_ref[...]
    out1 = dinv_ref[...] * agg + b_ref[...]
    h2 = jnp.dot(out1, w_ref[...], preferred_element_type=jnp.float32)
    y2_ref[...] = h2 * dinv_ref[...]


_tc_mid = pl.pallas_call(
    _tc_mid_body,
    grid=(_GRID,),
    in_specs=[
        pl.BlockSpec((_NC, _BR, _D), lambda i: (0, i, 0)),
        pl.BlockSpec((_BR, _D), lambda i: (i, 0)),
        pl.BlockSpec((_BR, 1), lambda i: (i, 0)),
        pl.BlockSpec((1, _D), lambda i: (0, 0)),
        pl.BlockSpec((_D, _D), lambda i: (0, 0)),
    ],
    out_specs=pl.BlockSpec((_BR, _D), lambda i: (i, 0)),
    out_shape=jax.ShapeDtypeStruct((_N, _D), jnp.float32),
)


def _tc_fin_body(q_ref, y2_ref, dinv_ref, b_ref, o_ref):
    z = dinv_ref[...] * (q_ref[0] + q_ref[1] + y2_ref[...]) + b_ref[...]
    o_ref[...] = jnp.where(z >= 0, z, 0.1 * z)


_tc_fin = pl.pallas_call(
    _tc_fin_body,
    grid=(_GRID,),
    in_specs=[
        pl.BlockSpec((_NC, _BR, _D), lambda i: (0, i, 0)),
        pl.BlockSpec((_BR, _D), lambda i: (i, 0)),
        pl.BlockSpec((_BR, 1), lambda i: (i, 0)),
        pl.BlockSpec((1, _D), lambda i: (0, 0)),
    ],
    out_specs=pl.BlockSpec((_BR, _D), lambda i: (i, 0)),
    out_shape=jax.ShapeDtypeStruct((_N, _D), jnp.float32),
)


@jax.jit
def _run(x, edge_index, W1, b1, W2, b2):
    src4 = edge_index[0].reshape(_NCHUNKS, 1, _K)
    dst4 = edge_index[1].reshape(_NCHUNKS, 1, _K)
    cnt = _sc_count(dst4).reshape(_NC, _NP, 1)
    y1, dinv = _tc_prep(x, W1, cnt)
    p = _sc_agg(y1, src4, dst4)
    y2 = _tc_mid(p, y1, dinv, b1.reshape(1, _D), W2)
    q = _sc_agg(y2, src4, dst4)
    return _tc_fin(q, y2, dinv, b2.reshape(1, _D))


def kernel(x, edge_index, W1, b1, W2, b2):
    return _run(x, edge_index, W1, b1, W2, b2)
